# baseline (speedup 1.0000x reference)
import dataclasses
import functools
import math

import jax
import jax.numpy as jnp
import numpy as np
from jax.experimental import pallas as pl
from jax.experimental.pallas import tpu as pltpu
from jax.experimental.pallas import tpu_sc as plsc

_CUTOFF = 10.0
_NG = 50
_NI = 6
_D = 128
_GRAPHS = 16

_NC = 2
_NS = 16
_NW = _NC * _NS

_CH = 80


def _ssp(v):
    return jax.nn.softplus(v) - math.log(2.0)


def _mm_body(x_ref, w_ref, o_ref):
    o_ref[...] = jnp.dot(x_ref[...], w_ref[...],
                         preferred_element_type=jnp.float32)


def _filter_body(d_ref, w1_ref, b1_ref, w2_ref, b2_ref, o_ref, *, width, coeff):
    d = d_ref[0, 0, :]
    offsets = (jax.lax.broadcasted_iota(jnp.int32, (1, _NG), 1)
               .astype(jnp.float32) * width)
    f = jnp.exp(coeff * (d[:, None] - offsets) ** 2)
    t = _ssp(jnp.dot(f, w1_ref[...], preferred_element_type=jnp.float32)
             + b1_ref[...])
    w = jnp.dot(t, w2_ref[...], preferred_element_type=jnp.float32) + b2_ref[...]
    c = 0.5 * (jnp.cos(d * (math.pi / _CUTOFF)) + 1.0)
    c = c * (d < _CUTOFF).astype(jnp.float32)
    o_ref[...] = (w * c[:, None]).astype(jnp.bfloat16)


def _post_body(aggp_ref, h_ref, fw_ref, fb_ref, iw_ref, ib_ref, o_ref):
    agg = aggp_ref[0] + aggp_ref[1]
    v = _ssp(jnp.dot(agg, fw_ref[...], preferred_element_type=jnp.float32)
             + fb_ref[...])
    v = jnp.dot(v, iw_ref[...], preferred_element_type=jnp.float32) + ib_ref[...]
    o_ref[...] = h_ref[...] + v


def _embed_body(x_ref, lw_ref, nw_ref, h_ref, hf_ref):
    h = jnp.dot(x_ref[...], lw_ref[...], preferred_element_type=jnp.float32)
    h_ref[...] = h
    hf_ref[...] = jnp.dot(h, nw_ref[...], preferred_element_type=jnp.float32)


def _postnext_body(aggp_ref, h_ref, fw_ref, fb_ref, iw_ref, ib_ref, nw_ref,
                   o_ref, hf_ref):
    agg = aggp_ref[0] + aggp_ref[1]
    v = _ssp(jnp.dot(agg, fw_ref[...], preferred_element_type=jnp.float32)
             + fb_ref[...])
    v = jnp.dot(v, iw_ref[...], preferred_element_type=jnp.float32) + ib_ref[...]
    hn = h_ref[...] + v
    o_ref[...] = hn
    hf_ref[...] = jnp.dot(hn, nw_ref[...], preferred_element_type=jnp.float32)


def _out_body(h_ref, w1_ref, b1_ref, w2_ref, b2_ref, w3_ref, b3_ref,
              batch_ref, o_ref):
    i = pl.program_id(0)
    o1 = _ssp(jnp.dot(h_ref[...], w1_ref[...],
                      preferred_element_type=jnp.float32) + b1_ref[...])
    o2 = _ssp(jnp.dot(o1, w2_ref[...],
                      preferred_element_type=jnp.float32) + b2_ref[...])
    o3 = jnp.sum(o2 * w3_ref[...], axis=1) + b3_ref[0, 0]
    b = batch_ref[0, 0, :]
    gids = jax.lax.broadcasted_iota(jnp.int32, (1, _GRAPHS), 1)
    m = (b[:, None] == gids).astype(jnp.float32)
    part = jnp.sum(m * o3[:, None], axis=0)

    @pl.when(i == 0)
    def _():
        o_ref[...] = jnp.zeros_like(o_ref)

    o_ref[0, :] += part


def _tc_matmul(x, w):
    m = x.shape[0]
    bm = 2000
    return pl.pallas_call(
        _mm_body,
        grid=(m // bm,),
        in_specs=[pl.BlockSpec((bm, x.shape[1]), lambda i: (i, 0)),
                  pl.BlockSpec(w.shape, lambda i: (0, 0))],
        out_specs=pl.BlockSpec((bm, w.shape[1]), lambda i: (i, 0)),
        out_shape=jax.ShapeDtypeStruct((m, w.shape[1]), jnp.float32),
    )(x, w)


def _tc_filter(dist3, w1, b1, w2, b2, n_edges, width, coeff):
    nblk, _, be = dist3.shape
    return pl.pallas_call(
        functools.partial(_filter_body, width=width, coeff=coeff),
        grid=(nblk,),
        in_specs=[pl.BlockSpec((1, 1, be), lambda i: (i, 0, 0)),
                  pl.BlockSpec(w1.shape, lambda i: (0, 0)),
                  pl.BlockSpec(b1.shape, lambda i: (0, 0)),
                  pl.BlockSpec(w2.shape, lambda i: (0, 0)),
                  pl.BlockSpec(b2.shape, lambda i: (0, 0))],
        out_specs=pl.BlockSpec((be, _D), lambda i: (i, 0)),
        out_shape=jax.ShapeDtypeStruct((n_edges, _D), jnp.bfloat16),
    )(dist3, w1, b1, w2, b2)


def _tc_embed(x, lw, nw):
    n = x.shape[0]
    bm = 2000
    return pl.pallas_call(
        _embed_body,
        grid=(n // bm,),
        in_specs=[pl.BlockSpec((bm, x.shape[1]), lambda i: (i, 0)),
                  pl.BlockSpec(lw.shape, lambda i: (0, 0)),
                  pl.BlockSpec(nw.shape, lambda i: (0, 0))],
        out_specs=[pl.BlockSpec((bm, _D), lambda i: (i, 0)),
                   pl.BlockSpec((bm, _D), lambda i: (i, 0))],
        out_shape=[jax.ShapeDtypeStruct((n, _D), jnp.float32),
                   jax.ShapeDtypeStruct((n, _D), jnp.float32)],
    )(x, lw, nw)


def _tc_postnext(aggp, h, fw, fb, iw, ib, nw):
    n = h.shape[0]
    bm = 2000
    return pl.pallas_call(
        _postnext_body,
        grid=(n // bm,),
        in_specs=[pl.BlockSpec((_NC, bm, _D), lambda i: (0, i, 0)),
                  pl.BlockSpec((bm, _D), lambda i: (i, 0)),
                  pl.BlockSpec(fw.shape, lambda i: (0, 0)),
                  pl.BlockSpec(fb.shape, lambda i: (0, 0)),
                  pl.BlockSpec(iw.shape, lambda i: (0, 0)),
                  pl.BlockSpec(ib.shape, lambda i: (0, 0)),
                  pl.BlockSpec(nw.shape, lambda i: (0, 0))],
        out_specs=[pl.BlockSpec((bm, _D), lambda i: (i, 0)),
                   pl.BlockSpec((bm, _D), lambda i: (i, 0))],
        out_shape=[jax.ShapeDtypeStruct((n, _D), jnp.float32),
                   jax.ShapeDtypeStruct((n, _D), jnp.float32)],
    )(aggp, h, fw, fb, iw, ib, nw)


def _tc_post(aggp, h, fw, fb, iw, ib):
    n = h.shape[0]
    bm = 2000
    return pl.pallas_call(
        _post_body,
        grid=(n // bm,),
        in_specs=[pl.BlockSpec((_NC, bm, _D), lambda i: (0, i, 0)),
                  pl.BlockSpec((bm, _D), lambda i: (i, 0)),
                  pl.BlockSpec(fw.shape, lambda i: (0, 0)),
                  pl.BlockSpec(fb.shape, lambda i: (0, 0)),
                  pl.BlockSpec(iw.shape, lambda i: (0, 0)),
                  pl.BlockSpec(ib.shape, lambda i: (0, 0))],
        out_specs=pl.BlockSpec((bm, _D), lambda i: (i, 0)),
        out_shape=jax.ShapeDtypeStruct((n, _D), jnp.float32),
    )(aggp, h, fw, fb, iw, ib)


def _tc_output(h, w1, b1, w2, b2, w3row, b3, batch3):
    n = h.shape[0]
    bm = 2000
    return pl.pallas_call(
        _out_body,
        grid=(n // bm,),
        in_specs=[pl.BlockSpec((bm, _D), lambda i: (i, 0)),
                  pl.BlockSpec(w1.shape, lambda i: (0, 0)),
                  pl.BlockSpec(b1.shape, lambda i: (0, 0)),
                  pl.BlockSpec(w2.shape, lambda i: (0, 0)),
                  pl.BlockSpec(b2.shape, lambda i: (0, 0)),
                  pl.BlockSpec(w3row.shape, lambda i: (0, 0)),
                  pl.BlockSpec(b3.shape, lambda i: (0, 0)),
                  pl.BlockSpec((1, 1, bm), lambda i: (i, 0, 0))],
        out_specs=pl.BlockSpec((1, _GRAPHS), lambda i: (0, 0)),
        out_shape=jax.ShapeDtypeStruct((1, _GRAPHS), jnp.float32),
    )(h, w1, b1, w2, b2, w3row, b3, batch3)


def _sc_aggregate(hf, w_edges, indi3, indj3):
    n_edges = w_edges.shape[0]
    ew = n_edges // _NW
    nchunk = ew // _CH
    npad = 10240
    rps = npad // _NS
    zr = _CH

    mesh = plsc.VectorSubcoreMesh(core_axis_name="c", subcore_axis_name="s")
    cp = pltpu.CompilerParams()
    if "needs_layout_passes" in pltpu.CompilerParams.__dataclass_fields__:
        cp = dataclasses.replace(cp, needs_layout_passes=False)

    @functools.partial(
        pl.kernel,
        out_type=jax.ShapeDtypeStruct((_NC, npad, _D), jnp.float32),
        mesh=mesh,
        compiler_params=cp,
        scratch_types=[
            pltpu.VMEM((_CH,), jnp.int32),
            pltpu.VMEM((_CH,), jnp.int32),
            pltpu.VMEM((_CH,), jnp.int32),
            pltpu.VMEM((_CH,), jnp.int32),
            pltpu.VMEM((_CH, _D), jnp.float32),
            pltpu.VMEM((_CH, _D // 2), jnp.int32),
            pltpu.VMEM((_CH, _D), jnp.float32),
            pltpu.VMEM((_CH, _D // 2), jnp.int32),
            pltpu.VMEM_SHARED((npad, _D), jnp.float32),
            pltpu.SemaphoreType.DMA,
            pltpu.SemaphoreType.DMA,
            pltpu.SemaphoreType.DMA,
            pltpu.SemaphoreType.DMA,
            pltpu.SemaphoreType.DMA,
            pltpu.SemaphoreType.DMA,
            pltpu.SemaphoreType.DMA,
            pltpu.SemaphoreType.DMA,
        ],
    )
    def k(hf_hbm, w_hbm, indi_hbm, indj_hbm, out_hbm,
          ii_a, ij_a, ii_b, ij_b, g_a, w_a, g_b, w_b, acc_sh,
          sii_a, sij_a, sii_b, sij_b, sg_a, sw_a, sg_b, sw_b):
        c = jax.lax.axis_index("c")
        s = jax.lax.axis_index("s")
        wid = c * _NS + s
        base0 = wid * ew

        @pl.loop(0, zr)
        def _(r):
            for kk in range(_D // 16):
                g_a.at[r, pl.ds(kk * 16, 16)][...] = (
                    jnp.zeros((16,), jnp.float32))

        @pl.loop(0, rps, step=zr)
        def _(r0):
            pltpu.sync_copy(g_a, acc_sh.at[pl.ds(s * rps + r0, zr)])

        plsc.subcore_barrier()

        def idx_load(chl, ii, ij, sii, sij):
            base = base0 + chl * _CH
            pltpu.async_copy(indi_hbm.at[pl.ds(base, _CH)], ii, sii)
            pltpu.async_copy(indj_hbm.at[pl.ds(base, _CH)], ij, sij)

        def idx_wait(chl, ii, ij, sii, sij):
            base = base0 + chl * _CH
            pltpu.make_async_copy(indi_hbm.at[pl.ds(base, _CH)], ii, sii).wait()
            pltpu.make_async_copy(indj_hbm.at[pl.ds(base, _CH)], ij, sij).wait()

        def gw_start(chl, ij, g, w, sg, sw):
            pltpu.async_copy(hf_hbm.at[ij], g, sg)
            pltpu.async_copy(w_hbm.at[pl.ds(base0 + chl * _CH, _CH)], w, sw)

        def process(chl, ii, ij, g, w, sg, sw):
            pltpu.make_async_copy(hf_hbm.at[ij], g, sg).wait()
            pltpu.make_async_copy(
                w_hbm.at[pl.ds(base0 + chl * _CH, _CH)], w, sw).wait()

            @pl.loop(0, _CH)
            def _(e):
                for gg in range(_D // 32):
                    wi = w.at[e, pl.ds(gg * 16, 16)][...]
                    lo = pl.ds(gg * 32, 16)
                    hi = pl.ds(gg * 32 + 16, 16)
                    g.at[e, lo][...] = (g.at[e, lo][...]
                                        * plsc.bitcast(wi << 16, jnp.float32))
                    g.at[e, hi][...] = (g.at[e, hi][...]
                                        * plsc.bitcast(wi & -65536, jnp.float32))

            pltpu.sync_copy(g, acc_sh.at[ii], add=True)

        idx_load(0, ii_a, ij_a, sii_a, sij_a)
        idx_load(1, ii_b, ij_b, sii_b, sij_b)
        idx_wait(0, ii_a, ij_a, sii_a, sij_a)
        gw_start(0, ij_a, g_a, w_a, sg_a, sw_a)

        @pl.loop(0, (nchunk - 3) // 2)
        def _(p):
            c0 = 2 * p
            idx_wait(c0 + 1, ii_b, ij_b, sii_b, sij_b)
            gw_start(c0 + 1, ij_b, g_b, w_b, sg_b, sw_b)
            process(c0, ii_a, ij_a, g_a, w_a, sg_a, sw_a)
            idx_load(c0 + 2, ii_a, ij_a, sii_a, sij_a)
            process(c0 + 1, ii_b, ij_b, g_b, w_b, sg_b, sw_b)
            idx_load(c0 + 3, ii_b, ij_b, sii_b, sij_b)
            idx_wait(c0 + 2, ii_a, ij_a, sii_a, sij_a)
            gw_start(c0 + 2, ij_a, g_a, w_a, sg_a, sw_a)

        idx_wait(nchunk - 2, ii_b, ij_b, sii_b, sij_b)
        gw_start(nchunk - 2, ij_b, g_b, w_b, sg_b, sw_b)
        process(nchunk - 3, ii_a, ij_a, g_a, w_a, sg_a, sw_a)
        idx_load(nchunk - 1, ii_a, ij_a, sii_a, sij_a)
        process(nchunk - 2, ii_b, ij_b, g_b, w_b, sg_b, sw_b)
        idx_wait(nchunk - 1, ii_a, ij_a, sii_a, sij_a)
        gw_start(nchunk - 1, ij_a, g_a, w_a, sg_a, sw_a)
        process(nchunk - 1, ii_a, ij_a, g_a, w_a, sg_a, sw_a)

        plsc.subcore_barrier()

        @pl.loop(0, rps, step=zr)
        def _(r0):
            pltpu.sync_copy(acc_sh.at[pl.ds(s * rps + r0, zr)],
                            out_hbm.at[c, pl.ds(s * rps + r0, zr)])

    return k(hf, w_edges, indi3, indj3)


def kernel(x, dist, dist_index, batch, lin_W, filt1_W, filt1_b, filt2_W,
           filt2_b, in2f_W, f2out_W, f2out_b, int_lin_W, int_lin_b,
           out1_W, out1_b, out2_W, out2_b, out3_W, out3_b):
    n_nodes = x.shape[0]
    n_edges = dist.shape[0]

    ind_i = dist_index[0].astype(jnp.int32)
    ind_j = dist_index[1].astype(jnp.int32)

    be = 2560
    dist3 = dist.reshape(n_edges // be, 1, be)
    batch3 = batch.astype(jnp.int32).reshape(n_nodes // 2000, 1, 2000)

    width = _CUTOFF / (_NG - 1)
    coeff = -0.5 / (width * width)

    perm = np.empty((_D,), np.int32)
    for g in range(_D // 32):
        for k in range(16):
            perm[32 * g + 2 * k] = 32 * g + k
            perm[32 * g + 2 * k + 1] = 32 * g + 16 + k

    def _pairs_i32(a):
        return jax.lax.bitcast_convert_type(
            a.reshape(a.shape[0], _D // 2, 2), jnp.int32)

    h, hf = _tc_embed(x, lin_W, in2f_W[0])

    for t in range(_NI):
        w_e = _tc_filter(dist3, filt1_W[t], filt1_b[t].reshape(1, -1),
                         filt2_W[t][:, perm], filt2_b[t][perm].reshape(1, -1),
                         n_edges, width, coeff)
        aggp = _sc_aggregate(hf, _pairs_i32(w_e), ind_i, ind_j)
        if t + 1 < _NI:
            h, hf = _tc_postnext(aggp, h, f2out_W[t], f2out_b[t].reshape(1, -1),
                                 int_lin_W[t], int_lin_b[t].reshape(1, -1),
                                 in2f_W[t + 1])
        else:
            h = _tc_post(aggp, h, f2out_W[t], f2out_b[t].reshape(1, -1),
                         int_lin_W[t], int_lin_b[t].reshape(1, -1))

    pooled = _tc_output(h, out1_W, out1_b.reshape(1, -1),
                        out2_W, out2_b.reshape(1, -1),
                        out3_W.reshape(1, -1), out3_b.reshape(1, 1), batch3)
    return pooled.reshape(-1)


# speedup vs baseline: 3.4274x; 3.4274x over previous
import dataclasses
import functools
import math

import jax
import jax.numpy as jnp
import numpy as np
from jax.experimental import pallas as pl
from jax.experimental.pallas import tpu as pltpu
from jax.experimental.pallas import tpu_sc as plsc

_CUTOFF = 10.0
_NG = 50
_NI = 6
_D = 128
_GRAPHS = 16

_NC = 2
_NS = 16
_NW = _NC * _NS

_CH = 80


def _ssp(v):
    return jax.nn.softplus(v) - math.log(2.0)


def _mm_body(x_ref, w_ref, o_ref):
    o_ref[...] = jnp.dot(x_ref[...], w_ref[...],
                         preferred_element_type=jnp.float32)


def _filter_body(d_ref, w1_ref, b1_ref, w2_ref, b2_ref, o_ref, *, width, coeff):
    d = d_ref[0, 0, :]
    offsets = (jax.lax.broadcasted_iota(jnp.int32, (1, _NG), 1)
               .astype(jnp.float32) * width)
    f = jnp.exp(coeff * (d[:, None] - offsets) ** 2)
    t = _ssp(jnp.dot(f, w1_ref[...], preferred_element_type=jnp.float32)
             + b1_ref[...])
    w = jnp.dot(t, w2_ref[...], preferred_element_type=jnp.float32) + b2_ref[...]
    c = 0.5 * (jnp.cos(d * (math.pi / _CUTOFF)) + 1.0)
    c = c * (d < _CUTOFF).astype(jnp.float32)
    o_ref[...] = w * c[:, None]


def _post_body(aggp_ref, h_ref, fw_ref, fb_ref, iw_ref, ib_ref, o_ref):
    agg = aggp_ref[0] + aggp_ref[1]
    v = _ssp(jnp.dot(agg, fw_ref[...], preferred_element_type=jnp.float32)
             + fb_ref[...])
    v = jnp.dot(v, iw_ref[...], preferred_element_type=jnp.float32) + ib_ref[...]
    o_ref[...] = h_ref[...] + v


def _embed_body(x_ref, lw_ref, nw_ref, h_ref, hf_ref):
    h = jnp.dot(x_ref[...], lw_ref[...], preferred_element_type=jnp.float32)
    h_ref[...] = h
    hf_ref[...] = jnp.dot(h, nw_ref[...], preferred_element_type=jnp.float32)


def _postnext_body(aggp_ref, h_ref, fw_ref, fb_ref, iw_ref, ib_ref, nw_ref,
                   o_ref, hf_ref):
    agg = aggp_ref[0] + aggp_ref[1]
    v = _ssp(jnp.dot(agg, fw_ref[...], preferred_element_type=jnp.float32)
             + fb_ref[...])
    v = jnp.dot(v, iw_ref[...], preferred_element_type=jnp.float32) + ib_ref[...]
    hn = h_ref[...] + v
    o_ref[...] = hn
    hf_ref[...] = jnp.dot(hn, nw_ref[...], preferred_element_type=jnp.float32)


def _out_body(h_ref, w1_ref, b1_ref, w2_ref, b2_ref, w3_ref, b3_ref,
              batch_ref, o_ref):
    i = pl.program_id(0)
    o1 = _ssp(jnp.dot(h_ref[...], w1_ref[...],
                      preferred_element_type=jnp.float32) + b1_ref[...])
    o2 = _ssp(jnp.dot(o1, w2_ref[...],
                      preferred_element_type=jnp.float32) + b2_ref[...])
    o3 = jnp.sum(o2 * w3_ref[...], axis=1) + b3_ref[0, 0]
    b = batch_ref[0, 0, :]
    gids = jax.lax.broadcasted_iota(jnp.int32, (1, _GRAPHS), 1)
    m = (b[:, None] == gids).astype(jnp.float32)
    part = jnp.sum(m * o3[:, None], axis=0)

    @pl.when(i == 0)
    def _():
        o_ref[...] = jnp.zeros_like(o_ref)

    o_ref[0, :] += part


def _tc_matmul(x, w):
    m = x.shape[0]
    bm = 2000
    return pl.pallas_call(
        _mm_body,
        grid=(m // bm,),
        in_specs=[pl.BlockSpec((bm, x.shape[1]), lambda i: (i, 0)),
                  pl.BlockSpec(w.shape, lambda i: (0, 0))],
        out_specs=pl.BlockSpec((bm, w.shape[1]), lambda i: (i, 0)),
        out_shape=jax.ShapeDtypeStruct((m, w.shape[1]), jnp.float32),
    )(x, w)


def _tc_filter(dist3, w1, b1, w2, b2, n_edges, width, coeff):
    nblk, _, be = dist3.shape
    return pl.pallas_call(
        functools.partial(_filter_body, width=width, coeff=coeff),
        grid=(nblk,),
        in_specs=[pl.BlockSpec((1, 1, be), lambda i: (i, 0, 0)),
                  pl.BlockSpec(w1.shape, lambda i: (0, 0)),
                  pl.BlockSpec(b1.shape, lambda i: (0, 0)),
                  pl.BlockSpec(w2.shape, lambda i: (0, 0)),
                  pl.BlockSpec(b2.shape, lambda i: (0, 0))],
        out_specs=pl.BlockSpec((be, _D), lambda i: (i, 0)),
        out_shape=jax.ShapeDtypeStruct((n_edges, _D), jnp.float32),
    )(dist3, w1, b1, w2, b2)


def _tc_embed(x, lw, nw):
    n = x.shape[0]
    bm = 2000
    return pl.pallas_call(
        _embed_body,
        grid=(n // bm,),
        in_specs=[pl.BlockSpec((bm, x.shape[1]), lambda i: (i, 0)),
                  pl.BlockSpec(lw.shape, lambda i: (0, 0)),
                  pl.BlockSpec(nw.shape, lambda i: (0, 0))],
        out_specs=[pl.BlockSpec((bm, _D), lambda i: (i, 0)),
                   pl.BlockSpec((bm, _D), lambda i: (i, 0))],
        out_shape=[jax.ShapeDtypeStruct((n, _D), jnp.float32),
                   jax.ShapeDtypeStruct((n, _D), jnp.float32)],
    )(x, lw, nw)


def _tc_postnext(aggp, h, fw, fb, iw, ib, nw):
    n = h.shape[0]
    bm = 2000
    return pl.pallas_call(
        _postnext_body,
        grid=(n // bm,),
        in_specs=[pl.BlockSpec((_NC, bm, _D), lambda i: (0, i, 0)),
                  pl.BlockSpec((bm, _D), lambda i: (i, 0)),
                  pl.BlockSpec(fw.shape, lambda i: (0, 0)),
                  pl.BlockSpec(fb.shape, lambda i: (0, 0)),
                  pl.BlockSpec(iw.shape, lambda i: (0, 0)),
                  pl.BlockSpec(ib.shape, lambda i: (0, 0)),
                  pl.BlockSpec(nw.shape, lambda i: (0, 0))],
        out_specs=[pl.BlockSpec((bm, _D), lambda i: (i, 0)),
                   pl.BlockSpec((bm, _D), lambda i: (i, 0))],
        out_shape=[jax.ShapeDtypeStruct((n, _D), jnp.float32),
                   jax.ShapeDtypeStruct((n, _D), jnp.float32)],
    )(aggp, h, fw, fb, iw, ib, nw)


def _tc_post(aggp, h, fw, fb, iw, ib):
    n = h.shape[0]
    bm = 2000
    return pl.pallas_call(
        _post_body,
        grid=(n // bm,),
        in_specs=[pl.BlockSpec((_NC, bm, _D), lambda i: (0, i, 0)),
                  pl.BlockSpec((bm, _D), lambda i: (i, 0)),
                  pl.BlockSpec(fw.shape, lambda i: (0, 0)),
                  pl.BlockSpec(fb.shape, lambda i: (0, 0)),
                  pl.BlockSpec(iw.shape, lambda i: (0, 0)),
                  pl.BlockSpec(ib.shape, lambda i: (0, 0))],
        out_specs=pl.BlockSpec((bm, _D), lambda i: (i, 0)),
        out_shape=jax.ShapeDtypeStruct((n, _D), jnp.float32),
    )(aggp, h, fw, fb, iw, ib)


def _tc_output(h, w1, b1, w2, b2, w3row, b3, batch3):
    n = h.shape[0]
    bm = 2000
    return pl.pallas_call(
        _out_body,
        grid=(n // bm,),
        in_specs=[pl.BlockSpec((bm, _D), lambda i: (i, 0)),
                  pl.BlockSpec(w1.shape, lambda i: (0, 0)),
                  pl.BlockSpec(b1.shape, lambda i: (0, 0)),
                  pl.BlockSpec(w2.shape, lambda i: (0, 0)),
                  pl.BlockSpec(b2.shape, lambda i: (0, 0)),
                  pl.BlockSpec(w3row.shape, lambda i: (0, 0)),
                  pl.BlockSpec(b3.shape, lambda i: (0, 0)),
                  pl.BlockSpec((1, 1, bm), lambda i: (i, 0, 0))],
        out_specs=pl.BlockSpec((1, _GRAPHS), lambda i: (0, 0)),
        out_shape=jax.ShapeDtypeStruct((1, _GRAPHS), jnp.float32),
    )(h, w1, b1, w2, b2, w3row, b3, batch3)


def _sc_aggregate(hf, w_edges, indi3, indj3):
    n_edges = w_edges.shape[0]
    ew = n_edges // _NW
    nchunk = ew // _CH
    npad = 10240
    rps = npad // _NS
    zr = _CH

    mesh = plsc.VectorSubcoreMesh(core_axis_name="c", subcore_axis_name="s")

    @functools.partial(
        pl.kernel,
        out_type=jax.ShapeDtypeStruct((_NC, npad, _D), jnp.float32),
        mesh=mesh,
        scratch_types=[
            pltpu.VMEM((_CH,), jnp.int32),
            pltpu.VMEM((_CH,), jnp.int32),
            pltpu.VMEM((_CH,), jnp.int32),
            pltpu.VMEM((_CH,), jnp.int32),
            pltpu.VMEM((_CH, _D), jnp.float32),
            pltpu.VMEM((_CH, _D), jnp.float32),
            pltpu.VMEM((_CH, _D), jnp.float32),
            pltpu.VMEM((_CH, _D), jnp.float32),
            pltpu.VMEM_SHARED((npad, _D), jnp.float32),
            pltpu.SemaphoreType.DMA,
            pltpu.SemaphoreType.DMA,
            pltpu.SemaphoreType.DMA,
            pltpu.SemaphoreType.DMA,
            pltpu.SemaphoreType.DMA,
            pltpu.SemaphoreType.DMA,
            pltpu.SemaphoreType.DMA,
            pltpu.SemaphoreType.DMA,
        ],
    )
    def k(hf_hbm, w_hbm, indi_hbm, indj_hbm, out_hbm,
          ii_a, ij_a, ii_b, ij_b, g_a, w_a, g_b, w_b, acc_sh,
          sii_a, sij_a, sii_b, sij_b, sg_a, sw_a, sg_b, sw_b):
        c = jax.lax.axis_index("c")
        s = jax.lax.axis_index("s")
        wid = c * _NS + s
        base0 = wid * ew

        @pl.loop(0, zr)
        def _(r):
            for kk in range(_D // 16):
                g_a.at[pl.ds(r, 1), pl.ds(kk * 16, 16)][...] = (
                    jnp.zeros((1, 16), jnp.float32))

        @pl.loop(0, rps, step=zr)
        def _(r0):
            pltpu.sync_copy(g_a, acc_sh.at[pl.ds(s * rps + r0, zr)])

        plsc.subcore_barrier()

        def idx_load(chl, ii, ij, sii, sij):
            base = base0 + chl * _CH
            pltpu.async_copy(indi_hbm.at[pl.ds(base, _CH)], ii, sii)
            pltpu.async_copy(indj_hbm.at[pl.ds(base, _CH)], ij, sij)

        def idx_wait(chl, ii, ij, sii, sij):
            base = base0 + chl * _CH
            pltpu.make_async_copy(indi_hbm.at[pl.ds(base, _CH)], ii, sii).wait()
            pltpu.make_async_copy(indj_hbm.at[pl.ds(base, _CH)], ij, sij).wait()

        def gw_start(chl, ij, g, w, sg, sw):
            pltpu.async_copy(hf_hbm.at[ij], g, sg)
            pltpu.async_copy(w_hbm.at[pl.ds(base0 + chl * _CH, _CH)], w, sw)

        def process(chl, ii, ij, g, w, sg, sw):
            pltpu.make_async_copy(hf_hbm.at[ij], g, sg).wait()
            pltpu.make_async_copy(
                w_hbm.at[pl.ds(base0 + chl * _CH, _CH)], w, sw).wait()

            @pl.loop(0, _CH)
            def _(e):
                for kk in range(_D // 16):
                    slc = (pl.ds(e, 1), pl.ds(kk * 16, 16))
                    g.at[*slc][...] = g.at[*slc][...] * w.at[*slc][...]

            pltpu.sync_copy(g, acc_sh.at[ii], add=True)

        idx_load(0, ii_a, ij_a, sii_a, sij_a)
        idx_load(1, ii_b, ij_b, sii_b, sij_b)
        idx_wait(0, ii_a, ij_a, sii_a, sij_a)
        gw_start(0, ij_a, g_a, w_a, sg_a, sw_a)

        @pl.loop(0, (nchunk - 3) // 2)
        def _(p):
            c0 = 2 * p
            idx_wait(c0 + 1, ii_b, ij_b, sii_b, sij_b)
            gw_start(c0 + 1, ij_b, g_b, w_b, sg_b, sw_b)
            process(c0, ii_a, ij_a, g_a, w_a, sg_a, sw_a)
            idx_load(c0 + 2, ii_a, ij_a, sii_a, sij_a)
            process(c0 + 1, ii_b, ij_b, g_b, w_b, sg_b, sw_b)
            idx_load(c0 + 3, ii_b, ij_b, sii_b, sij_b)
            idx_wait(c0 + 2, ii_a, ij_a, sii_a, sij_a)
            gw_start(c0 + 2, ij_a, g_a, w_a, sg_a, sw_a)

        idx_wait(nchunk - 2, ii_b, ij_b, sii_b, sij_b)
        gw_start(nchunk - 2, ij_b, g_b, w_b, sg_b, sw_b)
        process(nchunk - 3, ii_a, ij_a, g_a, w_a, sg_a, sw_a)
        idx_load(nchunk - 1, ii_a, ij_a, sii_a, sij_a)
        process(nchunk - 2, ii_b, ij_b, g_b, w_b, sg_b, sw_b)
        idx_wait(nchunk - 1, ii_a, ij_a, sii_a, sij_a)
        gw_start(nchunk - 1, ij_a, g_a, w_a, sg_a, sw_a)
        process(nchunk - 1, ii_a, ij_a, g_a, w_a, sg_a, sw_a)

        plsc.subcore_barrier()

        @pl.loop(0, rps, step=zr)
        def _(r0):
            pltpu.sync_copy(acc_sh.at[pl.ds(s * rps + r0, zr)],
                            out_hbm.at[c, pl.ds(s * rps + r0, zr)])

    return k(hf, w_edges, indi3, indj3)


def kernel(x, dist, dist_index, batch, lin_W, filt1_W, filt1_b, filt2_W,
           filt2_b, in2f_W, f2out_W, f2out_b, int_lin_W, int_lin_b,
           out1_W, out1_b, out2_W, out2_b, out3_W, out3_b):
    n_nodes = x.shape[0]
    n_edges = dist.shape[0]

    ind_i = dist_index[0].astype(jnp.int32)
    ind_j = dist_index[1].astype(jnp.int32)

    be = 2560
    dist3 = dist.reshape(n_edges // be, 1, be)
    batch3 = batch.astype(jnp.int32).reshape(n_nodes // 2000, 1, 2000)

    width = _CUTOFF / (_NG - 1)
    coeff = -0.5 / (width * width)

    h, hf = _tc_embed(x, lin_W, in2f_W[0])

    for t in range(_NI):
        w_e = _tc_filter(dist3, filt1_W[t], filt1_b[t].reshape(1, -1),
                         filt2_W[t], filt2_b[t].reshape(1, -1),
                         n_edges, width, coeff)
        aggp = _sc_aggregate(hf, w_e, ind_i, ind_j)
        if t + 1 < _NI:
            h, hf = _tc_postnext(aggp, h, f2out_W[t], f2out_b[t].reshape(1, -1),
                                 int_lin_W[t], int_lin_b[t].reshape(1, -1),
                                 in2f_W[t + 1])
        else:
            h = _tc_post(aggp, h, f2out_W[t], f2out_b[t].reshape(1, -1),
                         int_lin_W[t], int_lin_b[t].reshape(1, -1))

    pooled = _tc_output(h, out1_W, out1_b.reshape(1, -1),
                        out2_W, out2_b.reshape(1, -1),
                        out3_W.reshape(1, -1), out3_b.reshape(1, 1), batch3)
    return pooled.reshape(-1)


# speedup vs baseline: 3.4423x; 1.0043x over previous
import functools
import math

import jax
import jax.numpy as jnp
from jax.experimental import pallas as pl
from jax.experimental.pallas import tpu as pltpu
from jax.experimental.pallas import tpu_sc as plsc

_CUTOFF = 10.0
_NG = 50
_NI = 6
_D = 128
_GRAPHS = 16

_NC = 2
_NS = 16
_NW = _NC * _NS

_CH = 80


def _ssp(v):
    return jax.nn.softplus(v) - math.log(2.0)


def _mm_body(x_ref, w_ref, o_ref):
    o_ref[...] = jnp.dot(x_ref[...], w_ref[...],
                         preferred_element_type=jnp.float32)


def _filter_body(d_ref, w1_ref, b1_ref, w2_ref, b2_ref, o_ref, *, width, coeff):
    d = d_ref[0, 0, :]
    offsets = (jax.lax.broadcasted_iota(jnp.int32, (1, _NG), 1)
               .astype(jnp.float32) * width)
    f = jnp.exp(coeff * (d[:, None] - offsets) ** 2)
    t = _ssp(jnp.dot(f, w1_ref[...], preferred_element_type=jnp.float32)
             + b1_ref[...])
    w = jnp.dot(t, w2_ref[...], preferred_element_type=jnp.float32) + b2_ref[...]
    c = 0.5 * (jnp.cos(d * (math.pi / _CUTOFF)) + 1.0)
    c = c * (d < _CUTOFF).astype(jnp.float32)
    o_ref[...] = w * c[:, None]


def _embed_body(x_ref, lw_ref, nw_ref, h_ref, hf_ref):
    h = jnp.dot(x_ref[...], lw_ref[...], preferred_element_type=jnp.float32)
    h_ref[...] = h
    hf_ref[...] = jnp.dot(h, nw_ref[...], preferred_element_type=jnp.float32)


def _postnext_body(aggp_ref, h_ref, fw_ref, fb_ref, iw_ref, ib_ref, nw_ref,
                   o_ref, hf_ref):
    agg = aggp_ref[0] + aggp_ref[1]
    v = _ssp(jnp.dot(agg, fw_ref[...], preferred_element_type=jnp.float32)
             + fb_ref[...])
    v = jnp.dot(v, iw_ref[...], preferred_element_type=jnp.float32) + ib_ref[...]
    hn = h_ref[...] + v
    o_ref[...] = hn
    hf_ref[...] = jnp.dot(hn, nw_ref[...], preferred_element_type=jnp.float32)


def _finish_body(aggp_ref, h_ref, fw_ref, fb_ref, iw_ref, ib_ref,
                 w1_ref, b1_ref, w2_ref, b2_ref, w3_ref, b3_ref,
                 batch_ref, o_ref):
    i = pl.program_id(0)
    agg = aggp_ref[0] + aggp_ref[1]
    v = _ssp(jnp.dot(agg, fw_ref[...], preferred_element_type=jnp.float32)
             + fb_ref[...])
    v = jnp.dot(v, iw_ref[...], preferred_element_type=jnp.float32) + ib_ref[...]
    hn = h_ref[...] + v
    o1 = _ssp(jnp.dot(hn, w1_ref[...],
                      preferred_element_type=jnp.float32) + b1_ref[...])
    o2 = _ssp(jnp.dot(o1, w2_ref[...],
                      preferred_element_type=jnp.float32) + b2_ref[...])
    o3 = jnp.sum(o2 * w3_ref[...], axis=1) + b3_ref[0, 0]
    b = batch_ref[0, 0, :]
    gids = jax.lax.broadcasted_iota(jnp.int32, (1, _GRAPHS), 1)
    m = (b[:, None] == gids).astype(jnp.float32)
    part = jnp.sum(m * o3[:, None], axis=0)

    @pl.when(i == 0)
    def _():
        o_ref[...] = jnp.zeros_like(o_ref)

    o_ref[0, :] += part


def _tc_matmul(x, w):
    m = x.shape[0]
    bm = 2000
    return pl.pallas_call(
        _mm_body,
        grid=(m // bm,),
        in_specs=[pl.BlockSpec((bm, x.shape[1]), lambda i: (i, 0)),
                  pl.BlockSpec(w.shape, lambda i: (0, 0))],
        out_specs=pl.BlockSpec((bm, w.shape[1]), lambda i: (i, 0)),
        out_shape=jax.ShapeDtypeStruct((m, w.shape[1]), jnp.float32),
    )(x, w)


def _tc_filter(dist3, w1, b1, w2, b2, n_edges, width, coeff):
    nblk, _, be = dist3.shape
    return pl.pallas_call(
        functools.partial(_filter_body, width=width, coeff=coeff),
        grid=(nblk,),
        in_specs=[pl.BlockSpec((1, 1, be), lambda i: (i, 0, 0)),
                  pl.BlockSpec(w1.shape, lambda i: (0, 0)),
                  pl.BlockSpec(b1.shape, lambda i: (0, 0)),
                  pl.BlockSpec(w2.shape, lambda i: (0, 0)),
                  pl.BlockSpec(b2.shape, lambda i: (0, 0))],
        out_specs=pl.BlockSpec((be, _D), lambda i: (i, 0)),
        out_shape=jax.ShapeDtypeStruct((n_edges, _D), jnp.float32),
    )(dist3, w1, b1, w2, b2)


def _tc_embed(x, lw, nw):
    n = x.shape[0]
    bm = 2000
    return pl.pallas_call(
        _embed_body,
        grid=(n // bm,),
        in_specs=[pl.BlockSpec((bm, x.shape[1]), lambda i: (i, 0)),
                  pl.BlockSpec(lw.shape, lambda i: (0, 0)),
                  pl.BlockSpec(nw.shape, lambda i: (0, 0))],
        out_specs=[pl.BlockSpec((bm, _D), lambda i: (i, 0)),
                   pl.BlockSpec((bm, _D), lambda i: (i, 0))],
        out_shape=[jax.ShapeDtypeStruct((n, _D), jnp.float32),
                   jax.ShapeDtypeStruct((n, _D), jnp.float32)],
    )(x, lw, nw)


def _tc_postnext(aggp, h, fw, fb, iw, ib, nw):
    n = h.shape[0]
    bm = 2000
    return pl.pallas_call(
        _postnext_body,
        grid=(n // bm,),
        in_specs=[pl.BlockSpec((_NC, bm, _D), lambda i: (0, i, 0)),
                  pl.BlockSpec((bm, _D), lambda i: (i, 0)),
                  pl.BlockSpec(fw.shape, lambda i: (0, 0)),
                  pl.BlockSpec(fb.shape, lambda i: (0, 0)),
                  pl.BlockSpec(iw.shape, lambda i: (0, 0)),
                  pl.BlockSpec(ib.shape, lambda i: (0, 0)),
                  pl.BlockSpec(nw.shape, lambda i: (0, 0))],
        out_specs=[pl.BlockSpec((bm, _D), lambda i: (i, 0)),
                   pl.BlockSpec((bm, _D), lambda i: (i, 0))],
        out_shape=[jax.ShapeDtypeStruct((n, _D), jnp.float32),
                   jax.ShapeDtypeStruct((n, _D), jnp.float32)],
    )(aggp, h, fw, fb, iw, ib, nw)


def _tc_finish(aggp, h, fw, fb, iw, ib, w1, b1, w2, b2, w3row, b3, batch3):
    n = h.shape[0]
    bm = 2000
    return pl.pallas_call(
        _finish_body,
        grid=(n // bm,),
        in_specs=[pl.BlockSpec((_NC, bm, _D), lambda i: (0, i, 0)),
                  pl.BlockSpec((bm, _D), lambda i: (i, 0)),
                  pl.BlockSpec(fw.shape, lambda i: (0, 0)),
                  pl.BlockSpec(fb.shape, lambda i: (0, 0)),
                  pl.BlockSpec(iw.shape, lambda i: (0, 0)),
                  pl.BlockSpec(ib.shape, lambda i: (0, 0)),
                  pl.BlockSpec(w1.shape, lambda i: (0, 0)),
                  pl.BlockSpec(b1.shape, lambda i: (0, 0)),
                  pl.BlockSpec(w2.shape, lambda i: (0, 0)),
                  pl.BlockSpec(b2.shape, lambda i: (0, 0)),
                  pl.BlockSpec(w3row.shape, lambda i: (0, 0)),
                  pl.BlockSpec(b3.shape, lambda i: (0, 0)),
                  pl.BlockSpec((1, 1, bm), lambda i: (i, 0, 0))],
        out_specs=pl.BlockSpec((1, _GRAPHS), lambda i: (0, 0)),
        out_shape=jax.ShapeDtypeStruct((1, _GRAPHS), jnp.float32),
    )(aggp, h, fw, fb, iw, ib, w1, b1, w2, b2, w3row, b3, batch3)


def _sc_aggregate(hf, w_edges, indi3, indj3):
    n_edges = w_edges.shape[0]
    ew = n_edges // _NW
    nchunk = ew // _CH
    npad = 10240
    rps = npad // _NS
    zr = _CH

    mesh = plsc.VectorSubcoreMesh(core_axis_name="c", subcore_axis_name="s")

    @functools.partial(
        pl.kernel,
        out_type=jax.ShapeDtypeStruct((_NC, npad, _D), jnp.float32),
        mesh=mesh,
        scratch_types=[
            pltpu.VMEM((_CH,), jnp.int32),
            pltpu.VMEM((_CH,), jnp.int32),
            pltpu.VMEM((_CH,), jnp.int32),
            pltpu.VMEM((_CH,), jnp.int32),
            pltpu.VMEM((_CH, _D), jnp.float32),
            pltpu.VMEM((_CH, _D), jnp.float32),
            pltpu.VMEM((_CH, _D), jnp.float32),
            pltpu.VMEM((_CH, _D), jnp.float32),
            pltpu.VMEM_SHARED((npad, _D), jnp.float32),
            pltpu.SemaphoreType.DMA,
            pltpu.SemaphoreType.DMA,
            pltpu.SemaphoreType.DMA,
            pltpu.SemaphoreType.DMA,
            pltpu.SemaphoreType.DMA,
            pltpu.SemaphoreType.DMA,
            pltpu.SemaphoreType.DMA,
            pltpu.SemaphoreType.DMA,
        ],
    )
    def k(hf_hbm, w_hbm, indi_hbm, indj_hbm, out_hbm,
          ii_a, ij_a, ii_b, ij_b, g_a, w_a, g_b, w_b, acc_sh,
          sii_a, sij_a, sii_b, sij_b, sg_a, sw_a, sg_b, sw_b):
        c = jax.lax.axis_index("c")
        s = jax.lax.axis_index("s")
        wid = c * _NS + s
        base0 = wid * ew

        @pl.loop(0, zr)
        def _(r):
            for kk in range(_D // 16):
                g_a.at[pl.ds(r, 1), pl.ds(kk * 16, 16)][...] = (
                    jnp.zeros((1, 16), jnp.float32))

        @pl.loop(0, rps, step=zr)
        def _(r0):
            pltpu.sync_copy(g_a, acc_sh.at[pl.ds(s * rps + r0, zr)])

        plsc.subcore_barrier()

        def idx_load(chl, ii, ij, sii, sij):
            base = base0 + chl * _CH
            pltpu.async_copy(indi_hbm.at[pl.ds(base, _CH)], ii, sii)
            pltpu.async_copy(indj_hbm.at[pl.ds(base, _CH)], ij, sij)

        def idx_wait(chl, ii, ij, sii, sij):
            base = base0 + chl * _CH
            pltpu.make_async_copy(indi_hbm.at[pl.ds(base, _CH)], ii, sii).wait()
            pltpu.make_async_copy(indj_hbm.at[pl.ds(base, _CH)], ij, sij).wait()

        def gw_start(chl, ij, g, w, sg, sw):
            pltpu.async_copy(hf_hbm.at[ij], g, sg)
            pltpu.async_copy(w_hbm.at[pl.ds(base0 + chl * _CH, _CH)], w, sw)

        def process(chl, ii, ij, g, w, sg, sw):
            pltpu.make_async_copy(hf_hbm.at[ij], g, sg).wait()
            pltpu.make_async_copy(
                w_hbm.at[pl.ds(base0 + chl * _CH, _CH)], w, sw).wait()

            @pl.loop(0, _CH)
            def _(e):
                for kk in range(_D // 16):
                    slc = (pl.ds(e, 1), pl.ds(kk * 16, 16))
                    g.at[*slc][...] = g.at[*slc][...] * w.at[*slc][...]

            pltpu.sync_copy(g, acc_sh.at[ii], add=True)

        idx_load(0, ii_a, ij_a, sii_a, sij_a)
        idx_load(1, ii_b, ij_b, sii_b, sij_b)
        idx_wait(0, ii_a, ij_a, sii_a, sij_a)
        gw_start(0, ij_a, g_a, w_a, sg_a, sw_a)

        @pl.loop(0, (nchunk - 3) // 2)
        def _(p):
            c0 = 2 * p
            idx_wait(c0 + 1, ii_b, ij_b, sii_b, sij_b)
            gw_start(c0 + 1, ij_b, g_b, w_b, sg_b, sw_b)
            process(c0, ii_a, ij_a, g_a, w_a, sg_a, sw_a)
            idx_load(c0 + 2, ii_a, ij_a, sii_a, sij_a)
            process(c0 + 1, ii_b, ij_b, g_b, w_b, sg_b, sw_b)
            idx_load(c0 + 3, ii_b, ij_b, sii_b, sij_b)
            idx_wait(c0 + 2, ii_a, ij_a, sii_a, sij_a)
            gw_start(c0 + 2, ij_a, g_a, w_a, sg_a, sw_a)

        idx_wait(nchunk - 2, ii_b, ij_b, sii_b, sij_b)
        gw_start(nchunk - 2, ij_b, g_b, w_b, sg_b, sw_b)
        process(nchunk - 3, ii_a, ij_a, g_a, w_a, sg_a, sw_a)
        idx_load(nchunk - 1, ii_a, ij_a, sii_a, sij_a)
        process(nchunk - 2, ii_b, ij_b, g_b, w_b, sg_b, sw_b)
        idx_wait(nchunk - 1, ii_a, ij_a, sii_a, sij_a)
        gw_start(nchunk - 1, ij_a, g_a, w_a, sg_a, sw_a)
        process(nchunk - 1, ii_a, ij_a, g_a, w_a, sg_a, sw_a)

        plsc.subcore_barrier()

        @pl.loop(0, rps, step=zr)
        def _(r0):
            pltpu.sync_copy(acc_sh.at[pl.ds(s * rps + r0, zr)],
                            out_hbm.at[c, pl.ds(s * rps + r0, zr)])

    return k(hf, w_edges, indi3, indj3)


def kernel(x, dist, dist_index, batch, lin_W, filt1_W, filt1_b, filt2_W,
           filt2_b, in2f_W, f2out_W, f2out_b, int_lin_W, int_lin_b,
           out1_W, out1_b, out2_W, out2_b, out3_W, out3_b):
    n_nodes = x.shape[0]
    n_edges = dist.shape[0]

    ind_i = dist_index[0].astype(jnp.int32)
    ind_j = dist_index[1].astype(jnp.int32)

    be = 2560
    dist3 = dist.reshape(n_edges // be, 1, be)
    batch3 = batch.astype(jnp.int32).reshape(n_nodes // 2000, 1, 2000)

    width = _CUTOFF / (_NG - 1)
    coeff = -0.5 / (width * width)

    h, hf = _tc_embed(x, lin_W, in2f_W[0])

    for t in range(_NI):
        w_e = _tc_filter(dist3, filt1_W[t], filt1_b[t].reshape(1, -1),
                         filt2_W[t], filt2_b[t].reshape(1, -1),
                         n_edges, width, coeff)
        aggp = _sc_aggregate(hf, w_e, ind_i, ind_j)
        if t + 1 < _NI:
            h, hf = _tc_postnext(aggp, h, f2out_W[t], f2out_b[t].reshape(1, -1),
                                 int_lin_W[t], int_lin_b[t].reshape(1, -1),
                                 in2f_W[t + 1])
        else:
            pooled = _tc_finish(aggp, h, f2out_W[t], f2out_b[t].reshape(1, -1),
                                int_lin_W[t], int_lin_b[t].reshape(1, -1),
                                out1_W, out1_b.reshape(1, -1),
                                out2_W, out2_b.reshape(1, -1),
                                out3_W.reshape(1, -1), out3_b.reshape(1, 1),
                                batch3)

    return pooled.reshape(-1)


# speedup vs baseline: 3.4459x; 1.0011x over previous
import functools
import math

import jax
import jax.numpy as jnp
from jax.experimental import pallas as pl
from jax.experimental.pallas import tpu as pltpu
from jax.experimental.pallas import tpu_sc as plsc

_CUTOFF = 10.0
_NG = 50
_NI = 6
_D = 128
_GRAPHS = 16

_NC = 2
_NS = 16
_NW = _NC * _NS

_CH = 80


def _ssp(v):
    return jax.nn.softplus(v) - math.log(2.0)


def _mm_body(x_ref, w_ref, o_ref):
    o_ref[...] = jnp.dot(x_ref[...], w_ref[...],
                         preferred_element_type=jnp.float32)


def _filter_body(d_ref, w1_ref, b1_ref, w2_ref, b2_ref, o_ref, *, width, coeff):
    d = d_ref[0, 0, :]
    offsets = (jax.lax.broadcasted_iota(jnp.int32, (1, _NG), 1)
               .astype(jnp.float32) * width)
    f = jnp.exp(coeff * (d[:, None] - offsets) ** 2)
    t = _ssp(jnp.dot(f, w1_ref[...], preferred_element_type=jnp.float32)
             + b1_ref[...])
    w = jnp.dot(t, w2_ref[...], preferred_element_type=jnp.float32) + b2_ref[...]
    c = 0.5 * (jnp.cos(d * (math.pi / _CUTOFF)) + 1.0)
    c = c * (d < _CUTOFF).astype(jnp.float32)
    o_ref[...] = w * c[:, None]


def _embed_body(x_ref, lw_ref, nw_ref, h_ref, hf_ref):
    h = jnp.dot(x_ref[...], lw_ref[...], preferred_element_type=jnp.float32)
    h_ref[...] = h
    hf_ref[...] = jnp.dot(h, nw_ref[...], preferred_element_type=jnp.float32)


def _postnext_body(aggp_ref, h_ref, fw_ref, fb_ref, iw_ref, ib_ref, nw_ref,
                   o_ref, hf_ref):
    agg = aggp_ref[0] + aggp_ref[1]
    v = _ssp(jnp.dot(agg, fw_ref[...], preferred_element_type=jnp.float32)
             + fb_ref[...])
    v = jnp.dot(v, iw_ref[...], preferred_element_type=jnp.float32) + ib_ref[...]
    hn = h_ref[...] + v
    o_ref[...] = hn
    hf_ref[...] = jnp.dot(hn, nw_ref[...], preferred_element_type=jnp.float32)


def _finish_body(aggp_ref, h_ref, fw_ref, fb_ref, iw_ref, ib_ref,
                 w1_ref, b1_ref, w2_ref, b2_ref, w3_ref, b3_ref,
                 batch_ref, o_ref):
    i = pl.program_id(0)
    agg = aggp_ref[0] + aggp_ref[1]
    v = _ssp(jnp.dot(agg, fw_ref[...], preferred_element_type=jnp.float32)
             + fb_ref[...])
    v = jnp.dot(v, iw_ref[...], preferred_element_type=jnp.float32) + ib_ref[...]
    hn = h_ref[...] + v
    o1 = _ssp(jnp.dot(hn, w1_ref[...],
                      preferred_element_type=jnp.float32) + b1_ref[...])
    o2 = _ssp(jnp.dot(o1, w2_ref[...],
                      preferred_element_type=jnp.float32) + b2_ref[...])
    o3 = jnp.sum(o2 * w3_ref[...], axis=1) + b3_ref[0, 0]
    b = batch_ref[0, 0, :]
    gids = jax.lax.broadcasted_iota(jnp.int32, (1, _GRAPHS), 1)
    m = (b[:, None] == gids).astype(jnp.float32)
    part = jnp.sum(m * o3[:, None], axis=0)

    @pl.when(i == 0)
    def _():
        o_ref[...] = jnp.zeros_like(o_ref)

    o_ref[0, :] += part


def _tc_matmul(x, w):
    m = x.shape[0]
    bm = 2000
    return pl.pallas_call(
        _mm_body,
        grid=(m // bm,),
        in_specs=[pl.BlockSpec((bm, x.shape[1]), lambda i: (i, 0)),
                  pl.BlockSpec(w.shape, lambda i: (0, 0))],
        out_specs=pl.BlockSpec((bm, w.shape[1]), lambda i: (i, 0)),
        out_shape=jax.ShapeDtypeStruct((m, w.shape[1]), jnp.float32),
    )(x, w)


def _tc_filter(dist3, w1, b1, w2, b2, n_edges, width, coeff):
    nblk, _, be = dist3.shape
    return pl.pallas_call(
        functools.partial(_filter_body, width=width, coeff=coeff),
        grid=(nblk,),
        in_specs=[pl.BlockSpec((1, 1, be), lambda i: (i, 0, 0)),
                  pl.BlockSpec(w1.shape, lambda i: (0, 0)),
                  pl.BlockSpec(b1.shape, lambda i: (0, 0)),
                  pl.BlockSpec(w2.shape, lambda i: (0, 0)),
                  pl.BlockSpec(b2.shape, lambda i: (0, 0))],
        out_specs=pl.BlockSpec((be, _D), lambda i: (i, 0)),
        out_shape=jax.ShapeDtypeStruct((n_edges, _D), jnp.float32),
    )(dist3, w1, b1, w2, b2)


def _tc_embed(x, lw, nw):
    n = x.shape[0]
    bm = 2000
    return pl.pallas_call(
        _embed_body,
        grid=(n // bm,),
        in_specs=[pl.BlockSpec((bm, x.shape[1]), lambda i: (i, 0)),
                  pl.BlockSpec(lw.shape, lambda i: (0, 0)),
                  pl.BlockSpec(nw.shape, lambda i: (0, 0))],
        out_specs=[pl.BlockSpec((bm, _D), lambda i: (i, 0)),
                   pl.BlockSpec((bm, _D), lambda i: (i, 0))],
        out_shape=[jax.ShapeDtypeStruct((n, _D), jnp.float32),
                   jax.ShapeDtypeStruct((n, _D), jnp.float32)],
    )(x, lw, nw)


def _tc_postnext(aggp, h, fw, fb, iw, ib, nw):
    n = h.shape[0]
    bm = 2000
    return pl.pallas_call(
        _postnext_body,
        grid=(n // bm,),
        in_specs=[pl.BlockSpec((_NC, bm, _D), lambda i: (0, i, 0)),
                  pl.BlockSpec((bm, _D), lambda i: (i, 0)),
                  pl.BlockSpec(fw.shape, lambda i: (0, 0)),
                  pl.BlockSpec(fb.shape, lambda i: (0, 0)),
                  pl.BlockSpec(iw.shape, lambda i: (0, 0)),
                  pl.BlockSpec(ib.shape, lambda i: (0, 0)),
                  pl.BlockSpec(nw.shape, lambda i: (0, 0))],
        out_specs=[pl.BlockSpec((bm, _D), lambda i: (i, 0)),
                   pl.BlockSpec((bm, _D), lambda i: (i, 0))],
        out_shape=[jax.ShapeDtypeStruct((n, _D), jnp.float32),
                   jax.ShapeDtypeStruct((n, _D), jnp.float32)],
    )(aggp, h, fw, fb, iw, ib, nw)


def _tc_finish(aggp, h, fw, fb, iw, ib, w1, b1, w2, b2, w3row, b3, batch3):
    n = h.shape[0]
    bm = 2000
    return pl.pallas_call(
        _finish_body,
        grid=(n // bm,),
        in_specs=[pl.BlockSpec((_NC, bm, _D), lambda i: (0, i, 0)),
                  pl.BlockSpec((bm, _D), lambda i: (i, 0)),
                  pl.BlockSpec(fw.shape, lambda i: (0, 0)),
                  pl.BlockSpec(fb.shape, lambda i: (0, 0)),
                  pl.BlockSpec(iw.shape, lambda i: (0, 0)),
                  pl.BlockSpec(ib.shape, lambda i: (0, 0)),
                  pl.BlockSpec(w1.shape, lambda i: (0, 0)),
                  pl.BlockSpec(b1.shape, lambda i: (0, 0)),
                  pl.BlockSpec(w2.shape, lambda i: (0, 0)),
                  pl.BlockSpec(b2.shape, lambda i: (0, 0)),
                  pl.BlockSpec(w3row.shape, lambda i: (0, 0)),
                  pl.BlockSpec(b3.shape, lambda i: (0, 0)),
                  pl.BlockSpec((1, 1, bm), lambda i: (i, 0, 0))],
        out_specs=pl.BlockSpec((1, _GRAPHS), lambda i: (0, 0)),
        out_shape=jax.ShapeDtypeStruct((1, _GRAPHS), jnp.float32),
    )(aggp, h, fw, fb, iw, ib, w1, b1, w2, b2, w3row, b3, batch3)


def _sc_aggregate(hf, w_edges, indi3, indj3):
    n_edges = w_edges.shape[0]
    ew = n_edges // _NW
    nchunk = ew // _CH
    npad = 10240
    rps = npad // _NS
    zr = _CH

    mesh = plsc.VectorSubcoreMesh(core_axis_name="c", subcore_axis_name="s")

    @functools.partial(
        pl.kernel,
        out_type=jax.ShapeDtypeStruct((_NC, npad, _D), jnp.float32),
        mesh=mesh,
        scratch_types=[
            pltpu.VMEM((_CH,), jnp.int32),
            pltpu.VMEM((_CH,), jnp.int32),
            pltpu.VMEM((_CH,), jnp.int32),
            pltpu.VMEM((_CH,), jnp.int32),
            pltpu.VMEM((_CH, _D), jnp.float32),
            pltpu.VMEM((_CH, _D), jnp.float32),
            pltpu.VMEM((_CH, _D), jnp.float32),
            pltpu.VMEM((_CH, _D), jnp.float32),
            pltpu.VMEM_SHARED((npad, _D), jnp.float32),
            pltpu.SemaphoreType.DMA,
            pltpu.SemaphoreType.DMA,
            pltpu.SemaphoreType.DMA,
            pltpu.SemaphoreType.DMA,
            pltpu.SemaphoreType.DMA,
            pltpu.SemaphoreType.DMA,
            pltpu.SemaphoreType.DMA,
            pltpu.SemaphoreType.DMA,
        ],
    )
    def k(hf_hbm, w_hbm, indi_hbm, indj_hbm, out_hbm,
          ii_a, ij_a, ii_b, ij_b, g_a, w_a, g_b, w_b, acc_sh,
          sii_a, sij_a, sii_b, sij_b, sg_a, sw_a, sg_b, sw_b):
        c = jax.lax.axis_index("c")
        s = jax.lax.axis_index("s")
        wid = c * _NS + s
        base0 = wid * ew

        def idx_load(chl, ii, ij, sii, sij):
            base = base0 + chl * _CH
            pltpu.async_copy(indi_hbm.at[pl.ds(base, _CH)], ii, sii)
            pltpu.async_copy(indj_hbm.at[pl.ds(base, _CH)], ij, sij)

        def idx_wait(chl, ii, ij, sii, sij):
            base = base0 + chl * _CH
            pltpu.make_async_copy(indi_hbm.at[pl.ds(base, _CH)], ii, sii).wait()
            pltpu.make_async_copy(indj_hbm.at[pl.ds(base, _CH)], ij, sij).wait()

        def gw_start(chl, ij, g, w, sg, sw):
            pltpu.async_copy(hf_hbm.at[ij], g, sg)
            pltpu.async_copy(w_hbm.at[pl.ds(base0 + chl * _CH, _CH)], w, sw)

        def process(chl, ii, ij, g, w, sg, sw):
            pltpu.make_async_copy(hf_hbm.at[ij], g, sg).wait()
            pltpu.make_async_copy(
                w_hbm.at[pl.ds(base0 + chl * _CH, _CH)], w, sw).wait()

            @pl.loop(0, _CH)
            def _(e):
                for kk in range(_D // 16):
                    slc = (pl.ds(e, 1), pl.ds(kk * 16, 16))
                    g.at[*slc][...] = g.at[*slc][...] * w.at[*slc][...]

            pltpu.sync_copy(g, acc_sh.at[ii], add=True)

        idx_load(0, ii_a, ij_a, sii_a, sij_a)
        idx_load(1, ii_b, ij_b, sii_b, sij_b)

        @pl.loop(0, zr)
        def _(r):
            for kk in range(_D // 16):
                w_b.at[pl.ds(r, 1), pl.ds(kk * 16, 16)][...] = (
                    jnp.zeros((1, 16), jnp.float32))

        @pl.loop(0, rps, step=zr)
        def _(r0):
            pltpu.sync_copy(w_b, acc_sh.at[pl.ds(s * rps + r0, zr)])

        idx_wait(0, ii_a, ij_a, sii_a, sij_a)
        gw_start(0, ij_a, g_a, w_a, sg_a, sw_a)

        plsc.subcore_barrier()

        @pl.loop(0, (nchunk - 3) // 2)
        def _(p):
            c0 = 2 * p
            idx_wait(c0 + 1, ii_b, ij_b, sii_b, sij_b)
            gw_start(c0 + 1, ij_b, g_b, w_b, sg_b, sw_b)
            process(c0, ii_a, ij_a, g_a, w_a, sg_a, sw_a)
            idx_load(c0 + 2, ii_a, ij_a, sii_a, sij_a)
            process(c0 + 1, ii_b, ij_b, g_b, w_b, sg_b, sw_b)
            idx_load(c0 + 3, ii_b, ij_b, sii_b, sij_b)
            idx_wait(c0 + 2, ii_a, ij_a, sii_a, sij_a)
            gw_start(c0 + 2, ij_a, g_a, w_a, sg_a, sw_a)

        idx_wait(nchunk - 2, ii_b, ij_b, sii_b, sij_b)
        gw_start(nchunk - 2, ij_b, g_b, w_b, sg_b, sw_b)
        process(nchunk - 3, ii_a, ij_a, g_a, w_a, sg_a, sw_a)
        idx_load(nchunk - 1, ii_a, ij_a, sii_a, sij_a)
        process(nchunk - 2, ii_b, ij_b, g_b, w_b, sg_b, sw_b)
        idx_wait(nchunk - 1, ii_a, ij_a, sii_a, sij_a)
        gw_start(nchunk - 1, ij_a, g_a, w_a, sg_a, sw_a)
        process(nchunk - 1, ii_a, ij_a, g_a, w_a, sg_a, sw_a)

        plsc.subcore_barrier()

        @pl.loop(0, rps, step=zr)
        def _(r0):
            pltpu.sync_copy(acc_sh.at[pl.ds(s * rps + r0, zr)],
                            out_hbm.at[c, pl.ds(s * rps + r0, zr)])

    return k(hf, w_edges, indi3, indj3)


def kernel(x, dist, dist_index, batch, lin_W, filt1_W, filt1_b, filt2_W,
           filt2_b, in2f_W, f2out_W, f2out_b, int_lin_W, int_lin_b,
           out1_W, out1_b, out2_W, out2_b, out3_W, out3_b):
    n_nodes = x.shape[0]
    n_edges = dist.shape[0]

    ind_i = dist_index[0].astype(jnp.int32)
    ind_j = dist_index[1].astype(jnp.int32)

    be = 2560
    dist3 = dist.reshape(n_edges // be, 1, be)
    batch3 = batch.astype(jnp.int32).reshape(n_nodes // 2000, 1, 2000)

    width = _CUTOFF / (_NG - 1)
    coeff = -0.5 / (width * width)

    h, hf = _tc_embed(x, lin_W, in2f_W[0])

    for t in range(_NI):
        w_e = _tc_filter(dist3, filt1_W[t], filt1_b[t].reshape(1, -1),
                         filt2_W[t], filt2_b[t].reshape(1, -1),
                         n_edges, width, coeff)
        aggp = _sc_aggregate(hf, w_e, ind_i, ind_j)
        if t + 1 < _NI:
            h, hf = _tc_postnext(aggp, h, f2out_W[t], f2out_b[t].reshape(1, -1),
                                 int_lin_W[t], int_lin_b[t].reshape(1, -1),
                                 in2f_W[t + 1])
        else:
            pooled = _tc_finish(aggp, h, f2out_W[t], f2out_b[t].reshape(1, -1),
                                int_lin_W[t], int_lin_b[t].reshape(1, -1),
                                out1_W, out1_b.reshape(1, -1),
                                out2_W, out2_b.reshape(1, -1),
                                out3_W.reshape(1, -1), out3_b.reshape(1, 1),
                                batch3)

    return pooled.reshape(-1)
